# baseline (speedup 1.0000x reference)
import functools
import math

import jax
import jax.numpy as jnp
from jax import lax
from jax.experimental import pallas as pl
from jax.experimental.pallas import tpu as pltpu
from jax.experimental.pallas import tpu_sc as plsc

DIM = 64
SCALE = math.sqrt(DIM)

NC = 2
NS = 16
NW = NC * NS

W = 128
K = 4
MACRO = W * K


def _body(x_hbm, table_hbm, out_hbm, idx_v, rows_v, sem):
    n_rows_x = x_hbm.shape[0]
    rows_per_w = n_rows_x // NW
    macros = rows_per_w // K

    wid = lax.axis_index("s") * NC + lax.axis_index("c")
    xrow0 = wid * rows_per_w

    @pl.loop(0, macros)
    def _macro(m):
        xrow = xrow0 + m * K
        base = xrow * W

        pltpu.sync_copy(x_hbm.at[pl.ds(xrow, K)], idx_v)

        for j in range(K):
            pltpu.async_copy(
                table_hbm.at[idx_v.at[j]],
                rows_v.at[pl.ds(j * W, W)],
                sem,
            )
        for j in range(K):
            pltpu.make_async_copy(
                table_hbm.at[idx_v.at[j]],
                rows_v.at[pl.ds(j * W, W)],
                sem,
            ).wait()

        @pl.loop(0, MACRO)
        def _scale(i):
            for j in range(DIM // 16):
                sl = pl.ds(j * 16, 16)
                rows_v[i, sl] = rows_v[i, sl] * SCALE

        pltpu.sync_copy(rows_v, out_hbm.at[pl.ds(base, MACRO)])


def kernel(x, table):
    b, s = x.shape
    n = b * s
    x2 = x.reshape(n // W, W).astype(jnp.int32)

    grid_kernel = pl.kernel(
        functools.partial(_body),
        out_type=jax.ShapeDtypeStruct((n, DIM), jnp.float32),
        mesh=plsc.VectorSubcoreMesh(
            core_axis_name="c", subcore_axis_name="s",
            num_cores=NC, num_subcores=NS,
        ),
        scratch_types=[
            pltpu.VMEM((K, W), jnp.int32),
            pltpu.VMEM((MACRO, DIM), jnp.float32),
            pltpu.SemaphoreType.DMA,
        ],
        compiler_params=pltpu.CompilerParams(use_tc_tiling_on_sc=False),
    )
    out = grid_kernel(x2, table)
    return out.reshape(b, s, DIM)


# speedup vs baseline: 1.1296x; 1.1296x over previous
import functools
import math

import jax
import jax.numpy as jnp
from jax import lax
from jax.experimental import pallas as pl
from jax.experimental.pallas import tpu as pltpu
from jax.experimental.pallas import tpu_sc as plsc

DIM = 64
SCALE = math.sqrt(DIM)

NC = 2
NS = 16
NW = NC * NS

W = 128
K = 4
MACRO = W * K
NBUF = 2


def _body(x_hbm, table_hbm, out_hbm, idx_v, rows_v, gsem, ssem):
    n_rows_x = x_hbm.shape[0]
    rows_per_w = n_rows_x // NW
    macros = rows_per_w // K

    wid = lax.axis_index("s") * NC + lax.axis_index("c")
    xrow0 = wid * rows_per_w

    def stage_and_fire(m, b):
        xrow = xrow0 + m * K
        pltpu.sync_copy(x_hbm.at[pl.ds(xrow, K)], idx_v.at[b])
        for j in range(K):
            pltpu.async_copy(
                table_hbm.at[idx_v.at[b, j]],
                rows_v.at[b, pl.ds(j * W, W)],
                gsem[b],
            )

    def drain_scale_store(m, b):
        xrow = xrow0 + m * K
        for j in range(K):
            pltpu.make_async_copy(
                table_hbm.at[idx_v.at[b, j]],
                rows_v.at[b, pl.ds(j * W, W)],
                gsem[b],
            ).wait()

        @pl.loop(0, MACRO, unroll=8)
        def _scale(i):
            for j in range(DIM // 16):
                sl = pl.ds(j * 16, 16)
                rows_v[b, i, sl] = rows_v[b, i, sl] * SCALE

        pltpu.async_copy(rows_v.at[b], out_hbm.at[pl.ds(xrow * W, MACRO)], ssem[b])

    def wait_store(m, b):
        xrow = xrow0 + m * K
        pltpu.make_async_copy(
            rows_v.at[b], out_hbm.at[pl.ds(xrow * W, MACRO)], ssem[b]
        ).wait()

    stage_and_fire(0, 0)

    @pl.loop(0, macros, step=NBUF)
    def _macro(m0):
        for b in range(NBUF):
            m = m0 + b
            nxt = m + 1
            nb = (b + 1) % NBUF

            @pl.when(nxt < macros)
            def _fire_next():
                @pl.when(m >= 1)
                def _():
                    wait_store(m - 1, nb)
                stage_and_fire(nxt, nb)

            drain_scale_store(m, b)

    wait_store(macros - 1, (macros - 1) % NBUF)


def kernel(x, table):
    b, s = x.shape
    n = b * s
    x2 = x.reshape(n // W, W).astype(jnp.int32)

    grid_kernel = pl.kernel(
        functools.partial(_body),
        out_type=jax.ShapeDtypeStruct((n, DIM), jnp.float32),
        mesh=plsc.VectorSubcoreMesh(
            core_axis_name="c", subcore_axis_name="s",
            num_cores=NC, num_subcores=NS,
        ),
        scratch_types=[
            pltpu.VMEM((NBUF, K, W), jnp.int32),
            pltpu.VMEM((NBUF, MACRO, DIM), jnp.float32),
            [pltpu.SemaphoreType.DMA] * NBUF,
            [pltpu.SemaphoreType.DMA] * NBUF,
        ],
        compiler_params=pltpu.CompilerParams(use_tc_tiling_on_sc=False),
    )
    out = grid_kernel(x2, table)
    return out.reshape(b, s, DIM)
